# baseline (speedup 1.0000x reference)
import functools

import jax
import jax.numpy as jnp
from jax import lax
from jax.experimental import pallas as pl
from jax.experimental.pallas import tpu as pltpu
from jax.experimental.pallas import tpu_sc as plsc

_TOKENS = 32768
_EMB = 768
_E = 8
_BLK = 4096

_NC = 2
_NS = 16
_NW = _NC * _NS
_CHUNK = _TOKENS // _NW
_L = 16
_NEG_INF = float("-inf")


def _noisy_body(x_ref, w_ref, b_ref, snT_ref, noisyT_ref):
    x = x_ref[...]
    w = w_ref[...]
    acc = jnp.dot(x, w, preferred_element_type=jnp.float32)
    accT = acc.T + b_ref[...]
    logitsT = accT[:_E, :]
    nlogT = accT[_E:, :]
    softplus = jnp.maximum(nlogT, 0.0) + jnp.log1p(jnp.exp(-jnp.abs(nlogT)))
    noisyT_ref[...] = logitsT + snT_ref[...] * softplus


def _tc_noisy(x, w, b, snT):
    grid = (_TOKENS // _BLK,)
    return pl.pallas_call(
        _noisy_body,
        grid=grid,
        in_specs=[
            pl.BlockSpec((_BLK, _EMB), lambda i: (i, 0)),
            pl.BlockSpec((_EMB, 2 * _E), lambda i: (0, 0)),
            pl.BlockSpec((2 * _E, 1), lambda i: (0, 0)),
            pl.BlockSpec((_E, _BLK), lambda i: (0, i)),
        ],
        out_specs=pl.BlockSpec((_E, _BLK), lambda i: (0, i)),
        out_shape=jax.ShapeDtypeStruct((_E, _TOKENS), jnp.float32),
    )(x, w, b, snT)


@functools.partial(
    pl.kernel,
    mesh=plsc.VectorSubcoreMesh(core_axis_name="c", subcore_axis_name="s"),
    out_type=[
        jax.ShapeDtypeStruct((_TOKENS * _E,), jnp.float32),
        jax.ShapeDtypeStruct((_TOKENS * 2,), jnp.int32),
    ],
    scratch_types=[
        pltpu.VMEM((_E, _CHUNK), jnp.float32),
        pltpu.VMEM((_CHUNK * _E,), jnp.float32),
        pltpu.VMEM((_CHUNK * 2,), jnp.int32),
    ],
    compiler_params=pltpu.CompilerParams(needs_layout_passes=False),
)
def _sc_route(noisyT_hbm, out_hbm, idx_hbm, nT_v, out_v, idx_v):
    wid = lax.axis_index("s") * _NC + lax.axis_index("c")
    base = wid * _CHUNK
    pltpu.sync_copy(noisyT_hbm.at[:, pl.ds(base, _CHUNK)], nT_v)


    pltpu.sync_copy(out_v, out_hbm.at[pl.ds(base * _E, _CHUNK * _E)])
    pltpu.sync_copy(idx_v, idx_hbm.at[pl.ds(base * 2, _CHUNK * 2)])


def kernel(mha_out, Wg, bg, Wn, bn, topk):
    del topk
    w = jnp.concatenate([Wg, Wn], axis=0).T
    b = jnp.concatenate([bg, bn])[:, None]
    stdnormT = jax.random.normal(jax.random.key(42), (_TOKENS, _E), jnp.float32).T

    noisyT = _tc_noisy(mha_out, w, b, stdnormT)
    out_flat, idx_flat = _sc_route(noisyT)
    return (out_flat.reshape(_TOKENS, _E), idx_flat.reshape(_TOKENS, 2))


# speedup vs baseline: 2.4075x; 2.4075x over previous
import functools

import jax
import jax.numpy as jnp
from jax import lax
from jax.experimental import pallas as pl
from jax.experimental.pallas import tpu as pltpu

_TOKENS = 32768
_EMB = 768
_E = 8
_BLK = 2048


def _router_body(x_ref, w_ref, b_ref, snT_ref, outT_ref, idxT_ref):
    x = x_ref[...]
    w = w_ref[...]
    acc = jnp.dot(x, w, preferred_element_type=jnp.float32)
    accT = acc.T + b_ref[...]
    logitsT = accT[:_E, :]
    nlogT = accT[_E:, :]
    softplus = jnp.maximum(nlogT, 0.0) + jnp.log1p(jnp.exp(-jnp.abs(nlogT)))
    noisy = logitsT + snT_ref[...] * softplus

    ii = lax.broadcasted_iota(jnp.int32, noisy.shape, 0)
    m1 = jnp.max(noisy, axis=0, keepdims=True)
    i1 = jnp.min(jnp.where(noisy == m1, ii, _E), axis=0, keepdims=True)
    rest = jnp.where(ii == i1, -jnp.inf, noisy)
    m2 = jnp.max(rest, axis=0, keepdims=True)
    i2 = jnp.min(jnp.where(rest == m2, ii, _E), axis=0, keepdims=True)

    sel = (ii == i1) | (ii == i2)
    e = jnp.where(sel, jnp.exp(noisy - m1), 0.0)
    outT_ref[...] = e / jnp.sum(e, axis=0, keepdims=True)
    idxT_ref[...] = jnp.concatenate([i1, i2], axis=0)


def kernel(mha_out, Wg, bg, Wn, bn, topk):
    del topk
    w = jnp.concatenate([Wg, Wn], axis=0).T
    b = jnp.concatenate([bg, bn])[:, None]
    stdnormT = jax.random.normal(jax.random.key(42), (_TOKENS, _E), jnp.float32).T

    grid = (_TOKENS // _BLK,)
    outT, idxT = pl.pallas_call(
        _router_body,
        grid=grid,
        in_specs=[
            pl.BlockSpec((_BLK, _EMB), lambda i: (i, 0)),
            pl.BlockSpec((_EMB, 2 * _E), lambda i: (0, 0)),
            pl.BlockSpec((2 * _E, 1), lambda i: (0, 0)),
            pl.BlockSpec((_E, _BLK), lambda i: (0, i)),
        ],
        out_specs=[
            pl.BlockSpec((_E, _BLK), lambda i: (0, i)),
            pl.BlockSpec((2, _BLK), lambda i: (0, i)),
        ],
        out_shape=[
            jax.ShapeDtypeStruct((_E, _TOKENS), jnp.float32),
            jax.ShapeDtypeStruct((2, _TOKENS), jnp.int32),
        ],
    )(mha_out, w, b, stdnormT)
    return (outT.T, idxT.T)
